# Initial kernel scaffold; baseline (speedup 1.0000x reference)
import jax
import jax.numpy as jnp
from jax.experimental import pallas as pl


def kernel(centroids1, centroids2, features1, features2, W1, b1, g1, be1, W2, b2, g2, be2):
    raise NotImplementedError("write your pallas kernel here")


# speedup vs baseline: 16.6724x; 16.6724x over previous
import functools

import jax
import jax.numpy as jnp
from jax.experimental import pallas as pl

_BIG = 3.4e38


def _k1_body(c1n_ref, c2n_ref, c1_ref, c2_ref, f1_ref, f2_ref, w1_ref, b1_ref,
             h1_ref, s1_ref, q1_ref, *, n2, tn):
    b = pl.program_id(0)
    i = pl.program_id(1)
    c1b = c1_ref[0]
    c2b = c2_ref[0]
    f1b = f1_ref[0]
    f2b = f2_ref[0]

    cross = jax.lax.dot_general(
        c2b, c1b, (((0,), (0,)), ((), ())),
        preferred_element_type=jnp.float32)
    dist = (c1n_ref[0] + c2n_ref[0]) - 2.0 * cross

    rows = jax.lax.broadcasted_iota(jnp.int32, (n2, tn), 0)
    d = dist
    idxs = []
    recs = []
    for k in range(3):
        m = jnp.min(d, axis=0, keepdims=True)
        idxk = jnp.min(jnp.where(d == m, rows, n2), axis=0, keepdims=True)
        idxs.append(idxk)
        recs.append(1.0 / (m + 1e-8))
        if k < 2:
            d = jnp.where(rows == idxk, _BIG, d)
    rsum = recs[0] + recs[1] + recs[2]

    sel = jnp.zeros((n2, tn), jnp.float32)
    for k in range(3):
        sel = sel + jnp.where(rows == idxs[k], recs[k] / rsum, 0.0)

    interp = jax.lax.dot_general(
        f2b, sel, (((1,), (0,)), ((), ())),
        precision=jax.lax.Precision.HIGHEST,
        preferred_element_type=jnp.float32)
    feats = jnp.concatenate([f1b, interp], axis=0)
    h1 = jax.lax.dot_general(
        w1_ref[...], feats, (((1,), (0,)), ((), ())),
        preferred_element_type=jnp.float32) + b1_ref[...]
    h1_ref[0] = h1

    @pl.when((b == 0) & (i == 0))
    def _():
        s1_ref[...] = jnp.zeros_like(s1_ref)
        q1_ref[...] = jnp.zeros_like(q1_ref)

    s1_ref[...] += jnp.sum(h1, axis=1, keepdims=True)
    q1_ref[...] += jnp.sum(h1 * h1, axis=1, keepdims=True)


def _k2_body(h1_ref, s1_ref, q1_ref, g1_ref, be1_ref, w2_ref, b2_ref,
             h2_ref, s2_ref, q2_ref, *, count):
    b = pl.program_id(0)
    i = pl.program_id(1)
    mean = s1_ref[...] / count
    var = q1_ref[...] / count - mean * mean
    scale = g1_ref[...] * jax.lax.rsqrt(var + 1e-5)
    shift = be1_ref[...] - mean * scale
    a = jnp.maximum(h1_ref[0] * scale + shift, 0.0)
    h2 = jax.lax.dot_general(
        w2_ref[...], a, (((1,), (0,)), ((), ())),
        preferred_element_type=jnp.float32) + b2_ref[...]
    h2_ref[0] = h2

    @pl.when((b == 0) & (i == 0))
    def _():
        s2_ref[...] = jnp.zeros_like(s2_ref)
        q2_ref[...] = jnp.zeros_like(q2_ref)

    s2_ref[...] += jnp.sum(h2, axis=1, keepdims=True)
    q2_ref[...] += jnp.sum(h2 * h2, axis=1, keepdims=True)


def _k3_body(h2_ref, s2_ref, q2_ref, g2_ref, be2_ref, o_ref, *, count):
    mean = s2_ref[...] / count
    var = q2_ref[...] / count - mean * mean
    scale = g2_ref[...] * jax.lax.rsqrt(var + 1e-5)
    shift = be2_ref[...] - mean * scale
    o_ref[0] = jnp.maximum(h2_ref[0] * scale + shift, 0.0)


def kernel(centroids1, centroids2, features1, features2,
           W1, b1, g1, be1, W2, b2, g2, be2):
    B, _, N1 = centroids1.shape
    N2 = centroids2.shape[2]
    C1 = features1.shape[1]
    C2 = features2.shape[1]
    CO = W1.shape[0]
    TN = min(512, N1)
    NB = N1 // TN
    count = float(B * N1)

    b1c = b1.reshape(CO, 1)
    g1c = g1.reshape(CO, 1)
    be1c = be1.reshape(CO, 1)
    b2c = b2.reshape(CO, 1)
    g2c = g2.reshape(CO, 1)
    be2c = be2.reshape(CO, 1)

    grid = (B, NB)
    col_spec = pl.BlockSpec((CO, 1), lambda b, i: (0, 0))

    c1n = jnp.sum(jnp.transpose(centroids1, (0, 2, 1)) ** 2,
                  axis=-1).reshape(B, 1, N1)
    c2n = jnp.sum(jnp.transpose(centroids2, (0, 2, 1)) ** 2,
                  axis=-1).reshape(B, N2, 1)

    h1pre, s1, q1 = pl.pallas_call(
        functools.partial(_k1_body, n2=N2, tn=TN),
        grid=grid,
        in_specs=[
            pl.BlockSpec((1, 1, TN), lambda b, i: (b, 0, i)),
            pl.BlockSpec((1, N2, 1), lambda b, i: (b, 0, 0)),
            pl.BlockSpec((1, 3, TN), lambda b, i: (b, 0, i)),
            pl.BlockSpec((1, 3, N2), lambda b, i: (b, 0, 0)),
            pl.BlockSpec((1, C1, TN), lambda b, i: (b, 0, i)),
            pl.BlockSpec((1, C2, N2), lambda b, i: (b, 0, 0)),
            pl.BlockSpec((CO, C1 + C2), lambda b, i: (0, 0)),
            col_spec,
        ],
        out_specs=[
            pl.BlockSpec((1, CO, TN), lambda b, i: (b, 0, i)),
            col_spec,
            col_spec,
        ],
        out_shape=[
            jax.ShapeDtypeStruct((B, CO, N1), jnp.float32),
            jax.ShapeDtypeStruct((CO, 1), jnp.float32),
            jax.ShapeDtypeStruct((CO, 1), jnp.float32),
        ],
    )(c1n, c2n, centroids1, centroids2, features1, features2, W1, b1c)

    h2pre, s2, q2 = pl.pallas_call(
        functools.partial(_k2_body, count=count),
        grid=grid,
        in_specs=[
            pl.BlockSpec((1, CO, TN), lambda b, i: (b, 0, i)),
            col_spec, col_spec, col_spec, col_spec,
            pl.BlockSpec((CO, CO), lambda b, i: (0, 0)),
            col_spec,
        ],
        out_specs=[
            pl.BlockSpec((1, CO, TN), lambda b, i: (b, 0, i)),
            col_spec,
            col_spec,
        ],
        out_shape=[
            jax.ShapeDtypeStruct((B, CO, N1), jnp.float32),
            jax.ShapeDtypeStruct((CO, 1), jnp.float32),
            jax.ShapeDtypeStruct((CO, 1), jnp.float32),
        ],
    )(h1pre, s1, q1, g1c, be1c, W2, b2c)

    out = pl.pallas_call(
        functools.partial(_k3_body, count=count),
        grid=grid,
        in_specs=[
            pl.BlockSpec((1, CO, TN), lambda b, i: (b, 0, i)),
            col_spec, col_spec, col_spec, col_spec,
        ],
        out_specs=pl.BlockSpec((1, CO, TN), lambda b, i: (b, 0, i)),
        out_shape=jax.ShapeDtypeStruct((B, CO, N1), jnp.float32),
    )(h2pre, s2, q2, g2c, be2c)

    return out


# speedup vs baseline: 16.8743x; 1.0121x over previous
import functools

import jax
import jax.numpy as jnp
from jax.experimental import pallas as pl

_BIG = 3.4e38


def _k1_body(c1n_ref, c2n_ref, c1_ref, c2_ref, f1_ref, f2_ref, w1_ref, b1_ref,
             h1_ref, s1_ref, q1_ref, *, n2, tn):
    b = pl.program_id(0)
    i = pl.program_id(1)
    c1b = c1_ref[0]
    c2b = c2_ref[0]
    f1b = f1_ref[0]
    f2b = f2_ref[0]

    cross = jax.lax.dot_general(
        c2b, c1b, (((0,), (0,)), ((), ())),
        preferred_element_type=jnp.float32)
    dist = (c1n_ref[0] + c2n_ref[0]) - 2.0 * cross

    rows = jax.lax.broadcasted_iota(jnp.int32, (n2, tn), 0)
    d = dist
    recs = []
    sel = None
    for k in range(3):
        m = jnp.min(d, axis=0, keepdims=True)
        idxk = jnp.min(jnp.where(d == m, rows, n2), axis=0, keepdims=True)
        rk = 1.0 / (m + 1e-8)
        recs.append(rk)
        eqi = rows == idxk
        sel = jnp.where(eqi, rk, 0.0 if sel is None else sel)
        if k < 2:
            d = jnp.where(eqi, _BIG, d)
    rsum = recs[0] + recs[1] + recs[2]

    interp = jax.lax.dot_general(
        f2b, sel, (((1,), (0,)), ((), ())),
        precision=jax.lax.Precision.HIGHEST,
        preferred_element_type=jnp.float32) / rsum
    feats = jnp.concatenate([f1b, interp], axis=0)
    h1 = jax.lax.dot_general(
        w1_ref[...], feats, (((1,), (0,)), ((), ())),
        preferred_element_type=jnp.float32) + b1_ref[...]
    h1_ref[0] = h1

    @pl.when((b == 0) & (i == 0))
    def _():
        s1_ref[...] = jnp.zeros_like(s1_ref)
        q1_ref[...] = jnp.zeros_like(q1_ref)

    s1_ref[...] += jnp.sum(h1, axis=1, keepdims=True)
    q1_ref[...] += jnp.sum(h1 * h1, axis=1, keepdims=True)


def _k2_body(h1_ref, s1_ref, q1_ref, g1_ref, be1_ref, w2_ref, b2_ref,
             h2_ref, s2_ref, q2_ref, *, count):
    b = pl.program_id(0)
    i = pl.program_id(1)
    mean = s1_ref[...] / count
    var = q1_ref[...] / count - mean * mean
    scale = g1_ref[...] * jax.lax.rsqrt(var + 1e-5)
    shift = be1_ref[...] - mean * scale
    a = jnp.maximum(h1_ref[0] * scale + shift, 0.0)
    h2 = jax.lax.dot_general(
        w2_ref[...], a, (((1,), (0,)), ((), ())),
        preferred_element_type=jnp.float32) + b2_ref[...]
    h2_ref[0] = h2

    @pl.when((b == 0) & (i == 0))
    def _():
        s2_ref[...] = jnp.zeros_like(s2_ref)
        q2_ref[...] = jnp.zeros_like(q2_ref)

    s2_ref[...] += jnp.sum(h2, axis=1, keepdims=True)
    q2_ref[...] += jnp.sum(h2 * h2, axis=1, keepdims=True)


def _k3_body(h2_ref, s2_ref, q2_ref, g2_ref, be2_ref, o_ref, *, count):
    mean = s2_ref[...] / count
    var = q2_ref[...] / count - mean * mean
    scale = g2_ref[...] * jax.lax.rsqrt(var + 1e-5)
    shift = be2_ref[...] - mean * scale
    o_ref[0] = jnp.maximum(h2_ref[0] * scale + shift, 0.0)


def kernel(centroids1, centroids2, features1, features2,
           W1, b1, g1, be1, W2, b2, g2, be2):
    B, _, N1 = centroids1.shape
    N2 = centroids2.shape[2]
    C1 = features1.shape[1]
    C2 = features2.shape[1]
    CO = W1.shape[0]
    TN = min(512, N1)
    NB = N1 // TN
    count = float(B * N1)

    b1c = b1.reshape(CO, 1)
    g1c = g1.reshape(CO, 1)
    be1c = be1.reshape(CO, 1)
    b2c = b2.reshape(CO, 1)
    g2c = g2.reshape(CO, 1)
    be2c = be2.reshape(CO, 1)

    grid = (B, NB)
    col_spec = pl.BlockSpec((CO, 1), lambda b, i: (0, 0))

    c1n = jnp.sum(jnp.transpose(centroids1, (0, 2, 1)) ** 2,
                  axis=-1).reshape(B, 1, N1)
    c2n = jnp.sum(jnp.transpose(centroids2, (0, 2, 1)) ** 2,
                  axis=-1).reshape(B, N2, 1)

    h1pre, s1, q1 = pl.pallas_call(
        functools.partial(_k1_body, n2=N2, tn=TN),
        grid=grid,
        in_specs=[
            pl.BlockSpec((1, 1, TN), lambda b, i: (b, 0, i)),
            pl.BlockSpec((1, N2, 1), lambda b, i: (b, 0, 0)),
            pl.BlockSpec((1, 3, TN), lambda b, i: (b, 0, i)),
            pl.BlockSpec((1, 3, N2), lambda b, i: (b, 0, 0)),
            pl.BlockSpec((1, C1, TN), lambda b, i: (b, 0, i)),
            pl.BlockSpec((1, C2, N2), lambda b, i: (b, 0, 0)),
            pl.BlockSpec((CO, C1 + C2), lambda b, i: (0, 0)),
            col_spec,
        ],
        out_specs=[
            pl.BlockSpec((1, CO, TN), lambda b, i: (b, 0, i)),
            col_spec,
            col_spec,
        ],
        out_shape=[
            jax.ShapeDtypeStruct((B, CO, N1), jnp.float32),
            jax.ShapeDtypeStruct((CO, 1), jnp.float32),
            jax.ShapeDtypeStruct((CO, 1), jnp.float32),
        ],
    )(c1n, c2n, centroids1, centroids2, features1, features2, W1, b1c)

    h2pre, s2, q2 = pl.pallas_call(
        functools.partial(_k2_body, count=count),
        grid=grid,
        in_specs=[
            pl.BlockSpec((1, CO, TN), lambda b, i: (b, 0, i)),
            col_spec, col_spec, col_spec, col_spec,
            pl.BlockSpec((CO, CO), lambda b, i: (0, 0)),
            col_spec,
        ],
        out_specs=[
            pl.BlockSpec((1, CO, TN), lambda b, i: (b, 0, i)),
            col_spec,
            col_spec,
        ],
        out_shape=[
            jax.ShapeDtypeStruct((B, CO, N1), jnp.float32),
            jax.ShapeDtypeStruct((CO, 1), jnp.float32),
            jax.ShapeDtypeStruct((CO, 1), jnp.float32),
        ],
    )(h1pre, s1, q1, g1c, be1c, W2, b2c)

    out = pl.pallas_call(
        functools.partial(_k3_body, count=count),
        grid=grid,
        in_specs=[
            pl.BlockSpec((1, CO, TN), lambda b, i: (b, 0, i)),
            col_spec, col_spec, col_spec, col_spec,
        ],
        out_specs=pl.BlockSpec((1, CO, TN), lambda b, i: (b, 0, i)),
        out_shape=jax.ShapeDtypeStruct((B, CO, N1), jnp.float32),
    )(h2pre, s2, q2, g2c, be2c)

    return out


# speedup vs baseline: 18.5759x; 1.1008x over previous
import functools

import jax
import jax.numpy as jnp
from jax.experimental import pallas as pl

_BIG = 3.4e38


def _k1_body(c1n_ref, c2n_ref, c1_ref, c2_ref, f1_ref, f2_ref, w1_ref, b1_ref,
             h1_ref, s1_ref, q1_ref, *, n2, tn):
    b = pl.program_id(0)
    i = pl.program_id(1)
    c1b = c1_ref[0]
    c2b = c2_ref[0]
    f1b = f1_ref[0]
    f2b = f2_ref[0]

    cross = jax.lax.dot_general(
        c2b, c1b, (((0,), (0,)), ((), ())),
        preferred_element_type=jnp.float32)
    dist = (c1n_ref[0] + c2n_ref[0]) - 2.0 * cross

    rows = jax.lax.broadcasted_iota(jnp.int32, (n2, tn), 0)
    d = dist
    recs = []
    sel = None
    for k in range(3):
        m = jnp.min(d, axis=0, keepdims=True)
        idxk = jnp.min(jnp.where(d == m, rows, n2), axis=0, keepdims=True)
        rk = 1.0 / (m + 1e-8)
        recs.append(rk)
        eqi = rows == idxk
        sel = jnp.where(eqi, rk, 0.0 if sel is None else sel)
        if k < 2:
            d = jnp.where(eqi, _BIG, d)
    rsum = recs[0] + recs[1] + recs[2]

    interp = jax.lax.dot_general(
        f2b, sel, (((1,), (0,)), ((), ())),
        precision=jax.lax.Precision.HIGHEST,
        preferred_element_type=jnp.float32) / rsum
    feats = jnp.concatenate([f1b, interp], axis=0)
    h1 = jax.lax.dot_general(
        w1_ref[...], feats, (((1,), (0,)), ((), ())),
        preferred_element_type=jnp.float32) + b1_ref[...]
    h1_ref[0] = h1

    @pl.when((b == 0) & (i == 0))
    def _():
        s1_ref[...] = jnp.zeros_like(s1_ref)
        q1_ref[...] = jnp.zeros_like(q1_ref)

    s1_ref[...] += jnp.sum(h1, axis=1, keepdims=True)
    q1_ref[...] += jnp.sum(h1 * h1, axis=1, keepdims=True)


def _bn_affine(s_ref, q_ref, g_ref, be_ref, count):
    mean = s_ref[...] / count
    var = q_ref[...] / count - mean * mean
    scale = g_ref[...] * jax.lax.rsqrt(var + 1e-5)
    shift = be_ref[...] - mean * scale
    return scale, shift


def _layer2(h1_ref, s1_ref, q1_ref, g1_ref, be1_ref, w2_ref, b2_ref, count):
    scale, shift = _bn_affine(s1_ref, q1_ref, g1_ref, be1_ref, count)
    a = jnp.maximum(h1_ref[0] * scale + shift, 0.0)
    return jax.lax.dot_general(
        w2_ref[...], a, (((1,), (0,)), ((), ())),
        preferred_element_type=jnp.float32) + b2_ref[...]


def _k2_body(h1_ref, s1_ref, q1_ref, g1_ref, be1_ref, w2_ref, b2_ref,
             s2_ref, q2_ref, *, count):
    b = pl.program_id(0)
    i = pl.program_id(1)
    h2 = _layer2(h1_ref, s1_ref, q1_ref, g1_ref, be1_ref, w2_ref, b2_ref,
                 count)

    @pl.when((b == 0) & (i == 0))
    def _():
        s2_ref[...] = jnp.zeros_like(s2_ref)
        q2_ref[...] = jnp.zeros_like(q2_ref)

    s2_ref[...] += jnp.sum(h2, axis=1, keepdims=True)
    q2_ref[...] += jnp.sum(h2 * h2, axis=1, keepdims=True)


def _k3_body(h1_ref, s1_ref, q1_ref, g1_ref, be1_ref, w2_ref, b2_ref,
             s2_ref, q2_ref, g2_ref, be2_ref, o_ref, *, count):
    h2 = _layer2(h1_ref, s1_ref, q1_ref, g1_ref, be1_ref, w2_ref, b2_ref,
                 count)
    scale, shift = _bn_affine(s2_ref, q2_ref, g2_ref, be2_ref, count)
    o_ref[0] = jnp.maximum(h2 * scale + shift, 0.0)


def kernel(centroids1, centroids2, features1, features2,
           W1, b1, g1, be1, W2, b2, g2, be2):
    B, _, N1 = centroids1.shape
    N2 = centroids2.shape[2]
    C1 = features1.shape[1]
    C2 = features2.shape[1]
    CO = W1.shape[0]
    TN = min(512, N1)
    NB = N1 // TN
    count = float(B * N1)

    b1c = b1.reshape(CO, 1)
    g1c = g1.reshape(CO, 1)
    be1c = be1.reshape(CO, 1)
    b2c = b2.reshape(CO, 1)
    g2c = g2.reshape(CO, 1)
    be2c = be2.reshape(CO, 1)

    grid = (B, NB)
    col_spec = pl.BlockSpec((CO, 1), lambda b, i: (0, 0))

    c1n = jnp.sum(jnp.transpose(centroids1, (0, 2, 1)) ** 2,
                  axis=-1).reshape(B, 1, N1)
    c2n = jnp.sum(jnp.transpose(centroids2, (0, 2, 1)) ** 2,
                  axis=-1).reshape(B, N2, 1)

    h1pre, s1, q1 = pl.pallas_call(
        functools.partial(_k1_body, n2=N2, tn=TN),
        grid=grid,
        in_specs=[
            pl.BlockSpec((1, 1, TN), lambda b, i: (b, 0, i)),
            pl.BlockSpec((1, N2, 1), lambda b, i: (b, 0, 0)),
            pl.BlockSpec((1, 3, TN), lambda b, i: (b, 0, i)),
            pl.BlockSpec((1, 3, N2), lambda b, i: (b, 0, 0)),
            pl.BlockSpec((1, C1, TN), lambda b, i: (b, 0, i)),
            pl.BlockSpec((1, C2, N2), lambda b, i: (b, 0, 0)),
            pl.BlockSpec((CO, C1 + C2), lambda b, i: (0, 0)),
            col_spec,
        ],
        out_specs=[
            pl.BlockSpec((1, CO, TN), lambda b, i: (b, 0, i)),
            col_spec,
            col_spec,
        ],
        out_shape=[
            jax.ShapeDtypeStruct((B, CO, N1), jnp.float32),
            jax.ShapeDtypeStruct((CO, 1), jnp.float32),
            jax.ShapeDtypeStruct((CO, 1), jnp.float32),
        ],
    )(c1n, c2n, centroids1, centroids2, features1, features2, W1, b1c)

    TM = min(1024, N1)
    grid2 = (B, N1 // TM)
    blk = pl.BlockSpec((1, CO, TM), lambda b, i: (b, 0, i))
    w2_spec = pl.BlockSpec((CO, CO), lambda b, i: (0, 0))

    s2, q2 = pl.pallas_call(
        functools.partial(_k2_body, count=count),
        grid=grid2,
        in_specs=[blk, col_spec, col_spec, col_spec, col_spec, w2_spec,
                  col_spec],
        out_specs=[col_spec, col_spec],
        out_shape=[
            jax.ShapeDtypeStruct((CO, 1), jnp.float32),
            jax.ShapeDtypeStruct((CO, 1), jnp.float32),
        ],
    )(h1pre, s1, q1, g1c, be1c, W2, b2c)

    out = pl.pallas_call(
        functools.partial(_k3_body, count=count),
        grid=grid2,
        in_specs=[blk, col_spec, col_spec, col_spec, col_spec, w2_spec,
                  col_spec, col_spec, col_spec, col_spec, col_spec],
        out_specs=blk,
        out_shape=jax.ShapeDtypeStruct((B, CO, N1), jnp.float32),
    )(h1pre, s1, q1, g1c, be1c, W2, b2c, s2, q2, g2c, be2c)

    return out
